# baseline (speedup 1.0000x reference)
import jax
import jax.numpy as jnp
from jax import lax
from jax.experimental import pallas as pl
from jax.experimental.pallas import tpu as pltpu
from jax.experimental.pallas import tpu_sc as plsc

N = 10000
NPAD = 10240
E = 640000
D = 128
H = 4
F = 32

NC = 2
NS = 16
C = 80
EPT = E // NS
CPT = EPT // C
RPT = NPAD // NS
FIN = 80
ROW = 80


def _dense_tc(x_ref, g_ref, b_ref, w_ref, alr_ref, rw_ref,
              feat0_ref, feat1_ref, elr_ref, res_ref):
    x = x_ref[:]
    mean = jnp.mean(x, axis=0, keepdims=True)
    var = jnp.mean((x - mean) * (x - mean), axis=0, keepdims=True)
    o = (x - mean) / jnp.sqrt(var + 1e-5) * g_ref[:] + b_ref[:]
    o = jnp.maximum(o, 0.0)
    dn = (((1,), (1,)), ((), ()))
    feat = lax.dot_general(o, w_ref[:], dn, preferred_element_type=jnp.float32)
    feat0_ref[pl.ds(0, N), :] = feat[:, :64]
    feat1_ref[pl.ds(0, N), :] = feat[:, 64:]
    elr8 = lax.dot_general(feat, alr_ref[:], dn,
                           preferred_element_type=jnp.float32)
    elr_ref[0, pl.ds(0, N), :] = elr8[:, :4]
    elr_ref[1, pl.ds(0, N), :] = elr8[:, 4:]
    res = lax.dot_general(o, rw_ref[:], dn, preferred_element_type=jnp.float32)
    res_ref[0, pl.ds(0, N), :] = res[:, :64]
    res_ref[1, pl.ds(0, N), :] = res[:, 64:]


def _edge_sc(src_hbm, dst_hbm, feat0_hbm, feat1_hbm, elr_hbm, res_hbm,
             out_hbm, acc, elr_v, gb0, gb1, sb0, sb1,
             sv0, sv1, sv2, sv3, dv0, dv1, dv2, dv3, sx0, sx1,
             sem_g0, sem_g1, sem_s0, sem_s1,
             sem_i0, sem_i1, sem_i2, sem_i3):
    c = lax.axis_index("c")
    s = lax.axis_index("s")
    gbufs = [gb0, gb1]
    sbufs = [sb0, sb1]
    srcvs = [sv0, sv1, sv2, sv3]
    dstvs = [dv0, dv1, dv2, dv3]
    sdixs = [sx0, sx1]
    sem_gs = [sem_g0, sem_g1]
    sem_ss = [sem_s0, sem_s1]
    sem_is = [sem_i0, sem_i1, sem_i2, sem_i3]

    pltpu.sync_copy(elr_hbm.at[c], elr_v)

    def _zbody(i, carry):
        for g in range(5):
            sb0[i, pl.ds(g * 16, 16)] = jnp.zeros((16,), jnp.float32)
        return carry
    lax.fori_loop(0, C, _zbody, 0)
    for kk in range(8):
        pltpu.sync_copy(sb0.at[pl.ds(0, FIN)],
                        acc.at[pl.ds(s * RPT + kk * FIN, FIN)])
    def _zbody1(i, carry):
        for g in range(5):
            sb1[i, pl.ds(g * 16, 16)] = jnp.zeros((16,), jnp.float32)
        return carry
    lax.fori_loop(0, C, _zbody1, 0)
    plsc.subcore_barrier()

    iot = lax.iota(jnp.int32, 16)
    tbase = s * EPT

    def idx_issue(xv, slot):
        base = tbase + xv * C
        pltpu.async_copy(src_hbm.at[pl.ds(base, C)], srcvs[slot],
                         sem_is[slot])
        pltpu.async_copy(dst_hbm.at[pl.ds(base, C)], dstvs[slot],
                         sem_is[slot])

    def idx_wait(slot):
        pltpu.make_async_copy(src_hbm.at[pl.ds(0, C)], srcvs[slot],
                              sem_is[slot]).wait()
        pltpu.make_async_copy(dst_hbm.at[pl.ds(0, C)], dstvs[slot],
                              sem_is[slot]).wait()

    def gather_issue(slot, p):
        @pl.when(c == 0)
        def _():
            pltpu.async_copy(feat0_hbm.at[srcvs[slot]], gbufs[p], sem_gs[p])

        @pl.when(c == 1)
        def _():
            pltpu.async_copy(feat1_hbm.at[srcvs[slot]], gbufs[p], sem_gs[p])

    def gather_wait(slot, p):
        @pl.when(c == 0)
        def _():
            pltpu.make_async_copy(feat0_hbm.at[srcvs[slot]], gbufs[p],
                                  sem_gs[p]).wait()

        @pl.when(c == 1)
        def _():
            pltpu.make_async_copy(feat1_hbm.at[srcvs[slot]], gbufs[p],
                                  sem_gs[p]).wait()

    def scatter_issue(p):
        pltpu.async_copy(sbufs[p], acc.at[sdixs[p]], sem_ss[p], add=True)

    def scatter_wait(p):
        pltpu.make_async_copy(sbufs[p], acc.at[sdixs[p]], sem_ss[p]).wait()

    def body(xv, slot, p, s_wait):
        gather_wait(slot, p)
        if s_wait:
            scatter_wait(p)

        def _jbody(j16, carry):
            off = j16 * 16
            sv = srcvs[slot][pl.ds(off, 16)]
            dv = dstvs[slot][pl.ds(off, 16)]
            wvs = []
            for hl in range(2):
                a = plsc.load_gather(elr_v, [sv * 4 + hl])
                b = plsc.load_gather(elr_v, [dv * 4 + (2 + hl)])
                e = a + b
                e = jnp.where(e >= 0.0, e, 0.2 * e)
                wvs.append(jnp.exp(e))
            sdixs[p][pl.ds(off, 16)] = dv
            rowv = off + iot
            for col in range(64):
                cf = jnp.full((16,), col, jnp.int32)
                v = plsc.load_gather(gbufs[p], [rowv, cf])
                plsc.store_scatter(sbufs[p], [rowv, cf],
                                   v * wvs[col // 32])
            plsc.store_scatter(sbufs[p], [rowv, jnp.full((16,), 64, jnp.int32)],
                               wvs[0])
            plsc.store_scatter(sbufs[p], [rowv, jnp.full((16,), 65, jnp.int32)],
                               wvs[1])
            return carry
        lax.fori_loop(0, 5, _jbody, 0)
        scatter_issue(p)

    idx_issue(0, 0)
    idx_issue(1, 1)
    idx_issue(2, 2)
    idx_issue(3, 3)
    idx_wait(0)
    gather_issue(0, 0)
    idx_wait(1)
    gather_issue(1, 1)

    for x0 in (0, 1):
        body(x0, x0, x0, False)
        idx_issue(x0 + 4, x0)
        idx_wait(x0 + 2)
        gather_issue(x0 + 2, x0)

    def _quad(t, carry):
        for j in range(4):
            xv = 2 + 4 * t + j
            slot = (2 + j) % 4
            p = j % 2
            body(xv, slot, p, True)

            @pl.when(xv + 4 < CPT)
            def _():
                idx_issue(xv + 4, slot)
            idx_wait((slot + 2) % 4)
            gather_issue((slot + 2) % 4, p)
        return carry
    lax.fori_loop(0, (CPT - 4) // 4, _quad, 0)

    body(CPT - 2, 2, 0, True)
    body(CPT - 1, 3, 1, True)
    scatter_wait(0)
    scatter_wait(1)
    plsc.subcore_barrier()

    for kk in range(8):
        row0 = s * RPT + kk * FIN
        pltpu.sync_copy(acc.at[pl.ds(row0, FIN)], sb0.at[pl.ds(0, FIN)])
        pltpu.sync_copy(res_hbm.at[c, pl.ds(row0, FIN)],
                        gb0.at[pl.ds(0, FIN)])

        def _fin(i, carry):
            wv2 = sb0[i, pl.ds(64, 16)]
            d0 = wv2[0] + 1e-16
            d1 = wv2[1] + 1e-16
            for g in range(4):
                den = jnp.full((16,), d0 if g < 2 else d1, jnp.float32)
                v = sb0[i, pl.ds(g * 16, 16)] / den + gb0[i, pl.ds(g * 16, 16)]
                gb0[i, pl.ds(g * 16, 16)] = v
            return carry
        lax.fori_loop(0, FIN, _fin, 0)
        pltpu.sync_copy(gb0.at[pl.ds(0, FIN)], out_hbm.at[c, pl.ds(row0, FIN)])


def kernel(x, edge_index, bn_gamma, bn_beta, W, attn_l, attn_r, res_W):
    src = edge_index[0]
    dst = edge_index[1]

    alr = jnp.zeros((8, H * F), jnp.float32)
    for cc in range(2):
        for hl in range(2):
            h = 2 * cc + hl
            alr = alr.at[4 * cc + hl, h * F:(h + 1) * F].set(attn_l[h])
            alr = alr.at[4 * cc + 2 + hl, h * F:(h + 1) * F].set(attn_r[h])

    feat0, feat1, elr, res_s = pl.pallas_call(
        _dense_tc,
        out_shape=[
            jax.ShapeDtypeStruct((NPAD, 64), jnp.float32),
            jax.ShapeDtypeStruct((NPAD, 64), jnp.float32),
            jax.ShapeDtypeStruct((2, NPAD, 4), jnp.float32),
            jax.ShapeDtypeStruct((2, NPAD, 64), jnp.float32),
        ],
    )(x, bn_gamma, bn_beta, W, alr, res_W)

    mesh = plsc.VectorSubcoreMesh(core_axis_name="c", subcore_axis_name="s")
    out_s = pl.kernel(
        _edge_sc,
        out_type=jax.ShapeDtypeStruct((2, NPAD, 64), jnp.float32),
        mesh=mesh,
        compiler_params=pltpu.CompilerParams(needs_layout_passes=False,
                                             use_tc_tiling_on_sc=False),
        scratch_types=[
            pltpu.VMEM_SHARED((NPAD, ROW), jnp.float32),
            pltpu.VMEM((NPAD * 4,), jnp.float32),
            pltpu.VMEM((C, 64), jnp.float32),
            pltpu.VMEM((C, 64), jnp.float32),
            pltpu.VMEM((C, ROW), jnp.float32),
            pltpu.VMEM((C, ROW), jnp.float32),
            pltpu.VMEM((C,), jnp.int32),
            pltpu.VMEM((C,), jnp.int32),
            pltpu.VMEM((C,), jnp.int32),
            pltpu.VMEM((C,), jnp.int32),
            pltpu.VMEM((C,), jnp.int32),
            pltpu.VMEM((C,), jnp.int32),
            pltpu.VMEM((C,), jnp.int32),
            pltpu.VMEM((C,), jnp.int32),
            pltpu.VMEM((C,), jnp.int32),
            pltpu.VMEM((C,), jnp.int32),
            pltpu.SemaphoreType.DMA,
            pltpu.SemaphoreType.DMA,
            pltpu.SemaphoreType.DMA,
            pltpu.SemaphoreType.DMA,
            pltpu.SemaphoreType.DMA,
            pltpu.SemaphoreType.DMA,
            pltpu.SemaphoreType.DMA,
            pltpu.SemaphoreType.DMA,
        ],
    )(src, dst, feat0, feat1, elr.reshape(2, NPAD * 4), res_s)

    return jnp.concatenate([out_s[0, :N], out_s[1, :N]], axis=1).reshape(N, H, F)


# speedup vs baseline: 2.3799x; 2.3799x over previous
import jax
import jax.numpy as jnp
from jax import lax
from jax.experimental import pallas as pl
from jax.experimental.pallas import tpu as pltpu
from jax.experimental.pallas import tpu_sc as plsc

N = 10000
NPAD = 10240
E = 640000
D = 128
H = 4
F = 32

NC = 2
NS = 16
C = 80
EPT = E // NS
CPT = EPT // C
RPT = NPAD // NS
FIN = 80
ROW = 80


def _dense_tc(x_ref, g_ref, b_ref, w_ref, alr_ref, rw_ref,
              feat0_ref, feat1_ref, elr_ref, res_ref):
    x = x_ref[:]
    mean = jnp.mean(x, axis=0, keepdims=True)
    var = jnp.mean((x - mean) * (x - mean), axis=0, keepdims=True)
    o = (x - mean) / jnp.sqrt(var + 1e-5) * g_ref[:] + b_ref[:]
    o = jnp.maximum(o, 0.0)
    dn = (((1,), (1,)), ((), ()))
    feat = lax.dot_general(o, w_ref[:], dn, preferred_element_type=jnp.float32)
    feat0_ref[pl.ds(0, N), :] = feat[:, :64]
    feat1_ref[pl.ds(0, N), :] = feat[:, 64:]
    elr8 = lax.dot_general(feat, alr_ref[:], dn,
                           preferred_element_type=jnp.float32)
    elr_ref[0, pl.ds(0, N), :] = elr8[:, :4]
    elr_ref[1, pl.ds(0, N), :] = elr8[:, 4:]
    res = lax.dot_general(o, rw_ref[:], dn, preferred_element_type=jnp.float32)
    res_ref[0, pl.ds(0, N), :] = res[:, :64]
    res_ref[1, pl.ds(0, N), :] = res[:, 64:]


def _edge_sc(src_hbm, dst_hbm, feat0_hbm, feat1_hbm, elr_hbm, res_hbm,
             out_hbm, acc, elr_v, gb0, gb1, sb0, sb1,
             sv0, sv1, sv2, sv3, dv0, dv1, dv2, dv3, sx0, sx1,
             sem_g0, sem_g1, sem_s0, sem_s1,
             sem_i0, sem_i1, sem_i2, sem_i3):
    c = lax.axis_index("c")
    s = lax.axis_index("s")
    gbufs = [gb0, gb1]
    sbufs = [sb0, sb1]
    srcvs = [sv0, sv1, sv2, sv3]
    dstvs = [dv0, dv1, dv2, dv3]
    sdixs = [sx0, sx1]
    sem_gs = [sem_g0, sem_g1]
    sem_ss = [sem_s0, sem_s1]
    sem_is = [sem_i0, sem_i1, sem_i2, sem_i3]

    pltpu.sync_copy(elr_hbm.at[c], elr_v)

    def _zbody(i, carry):
        for g in range(5):
            sb0[i, pl.ds(g * 16, 16)] = jnp.zeros((16,), jnp.float32)
        return carry
    lax.fori_loop(0, C, _zbody, 0)
    for kk in range(8):
        pltpu.sync_copy(sb0.at[pl.ds(0, FIN)],
                        acc.at[pl.ds(s * RPT + kk * FIN, FIN)])
    def _zbody1(i, carry):
        for g in range(5):
            sb1[i, pl.ds(g * 16, 16)] = jnp.zeros((16,), jnp.float32)
        return carry
    lax.fori_loop(0, C, _zbody1, 0)
    plsc.subcore_barrier()

    iot = lax.iota(jnp.int32, 16)
    tbase = s * EPT

    def idx_issue(xv, slot):
        base = tbase + xv * C
        pltpu.async_copy(src_hbm.at[pl.ds(base, C)], srcvs[slot],
                         sem_is[slot])
        pltpu.async_copy(dst_hbm.at[pl.ds(base, C)], dstvs[slot],
                         sem_is[slot])

    def idx_wait(slot):
        pltpu.make_async_copy(src_hbm.at[pl.ds(0, C)], srcvs[slot],
                              sem_is[slot]).wait()
        pltpu.make_async_copy(dst_hbm.at[pl.ds(0, C)], dstvs[slot],
                              sem_is[slot]).wait()

    def gather_issue(slot, p):
        @pl.when(c == 0)
        def _():
            pltpu.async_copy(feat0_hbm.at[srcvs[slot]], gbufs[p], sem_gs[p])

        @pl.when(c == 1)
        def _():
            pltpu.async_copy(feat1_hbm.at[srcvs[slot]], gbufs[p], sem_gs[p])

    def gather_wait(slot, p):
        @pl.when(c == 0)
        def _():
            pltpu.make_async_copy(feat0_hbm.at[srcvs[slot]], gbufs[p],
                                  sem_gs[p]).wait()

        @pl.when(c == 1)
        def _():
            pltpu.make_async_copy(feat1_hbm.at[srcvs[slot]], gbufs[p],
                                  sem_gs[p]).wait()

    def scatter_issue(p):
        pltpu.async_copy(sbufs[p], acc.at[sdixs[p]], sem_ss[p], add=True)

    def scatter_wait(p):
        pltpu.make_async_copy(sbufs[p], acc.at[sdixs[p]], sem_ss[p]).wait()

    def body(xv, slot, p, s_wait):
        gather_wait(slot, p)
        if s_wait:
            scatter_wait(p)

        def _jbody(j16, carry):
            off = j16 * 16
            sv = srcvs[slot][pl.ds(off, 16)]
            dv = dstvs[slot][pl.ds(off, 16)]
            wvs = []
            for hl in range(2):
                a = plsc.load_gather(elr_v, [sv * 4 + hl])
                b = plsc.load_gather(elr_v, [dv * 4 + (2 + hl)])
                e = a + b
                e = jnp.where(e >= 0.0, e, 0.2 * e)
                wvs.append(jnp.exp(e))
            sdixs[p][pl.ds(off, 16)] = dv
            for l in range(16):
                row = off + l
                lful = jnp.full((16,), l, jnp.int32)
                w0b = jnp.take_along_axis(wvs[0], lful, axis=0)
                w1b = jnp.take_along_axis(wvs[1], lful, axis=0)
                for g in range(4):
                    wvb = w0b if g < 2 else w1b
                    sbufs[p][row, pl.ds(g * 16, 16)] = (
                        gbufs[p][row, pl.ds(g * 16, 16)] * wvb)
                w01 = jnp.where(iot == 0, w0b,
                                jnp.where(iot == 1, w1b, 0.0))
                sbufs[p][row, pl.ds(64, 16)] = w01
            return carry
        lax.fori_loop(0, 5, _jbody, 0)
        scatter_issue(p)

    idx_issue(0, 0)
    idx_issue(1, 1)
    idx_issue(2, 2)
    idx_issue(3, 3)
    idx_wait(0)
    gather_issue(0, 0)
    idx_wait(1)
    gather_issue(1, 1)

    for x0 in (0, 1):
        body(x0, x0, x0, False)
        idx_issue(x0 + 4, x0)
        idx_wait(x0 + 2)
        gather_issue(x0 + 2, x0)

    def _quad(t, carry):
        for j in range(4):
            xv = 2 + 4 * t + j
            slot = (2 + j) % 4
            p = j % 2
            body(xv, slot, p, True)

            @pl.when(xv + 4 < CPT)
            def _():
                idx_issue(xv + 4, slot)
            idx_wait((slot + 2) % 4)
            gather_issue((slot + 2) % 4, p)
        return carry
    lax.fori_loop(0, (CPT - 4) // 4, _quad, 0)

    body(CPT - 2, 2, 0, True)
    body(CPT - 1, 3, 1, True)
    scatter_wait(0)
    scatter_wait(1)
    plsc.subcore_barrier()

    for kk in range(8):
        row0 = s * RPT + kk * FIN
        pltpu.sync_copy(acc.at[pl.ds(row0, FIN)], sb0.at[pl.ds(0, FIN)])
        pltpu.sync_copy(res_hbm.at[c, pl.ds(row0, FIN)],
                        gb0.at[pl.ds(0, FIN)])

        def _fin(i, carry):
            wv2 = sb0[i, pl.ds(64, 16)]
            d0 = wv2[0] + 1e-16
            d1 = wv2[1] + 1e-16
            for g in range(4):
                den = jnp.full((16,), d0 if g < 2 else d1, jnp.float32)
                v = sb0[i, pl.ds(g * 16, 16)] / den + gb0[i, pl.ds(g * 16, 16)]
                gb0[i, pl.ds(g * 16, 16)] = v
            return carry
        lax.fori_loop(0, FIN, _fin, 0)
        pltpu.sync_copy(gb0.at[pl.ds(0, FIN)], out_hbm.at[c, pl.ds(row0, FIN)])


def kernel(x, edge_index, bn_gamma, bn_beta, W, attn_l, attn_r, res_W):
    src = edge_index[0]
    dst = edge_index[1]

    alr = jnp.zeros((8, H * F), jnp.float32)
    for cc in range(2):
        for hl in range(2):
            h = 2 * cc + hl
            alr = alr.at[4 * cc + hl, h * F:(h + 1) * F].set(attn_l[h])
            alr = alr.at[4 * cc + 2 + hl, h * F:(h + 1) * F].set(attn_r[h])

    feat0, feat1, elr, res_s = pl.pallas_call(
        _dense_tc,
        out_shape=[
            jax.ShapeDtypeStruct((NPAD, 64), jnp.float32),
            jax.ShapeDtypeStruct((NPAD, 64), jnp.float32),
            jax.ShapeDtypeStruct((2, NPAD, 4), jnp.float32),
            jax.ShapeDtypeStruct((2, NPAD, 64), jnp.float32),
        ],
    )(x, bn_gamma, bn_beta, W, alr, res_W)

    mesh = plsc.VectorSubcoreMesh(core_axis_name="c", subcore_axis_name="s")
    out_s = pl.kernel(
        _edge_sc,
        out_type=jax.ShapeDtypeStruct((2, NPAD, 64), jnp.float32),
        mesh=mesh,
        compiler_params=pltpu.CompilerParams(needs_layout_passes=False,
                                             use_tc_tiling_on_sc=False),
        scratch_types=[
            pltpu.VMEM_SHARED((NPAD, ROW), jnp.float32),
            pltpu.VMEM((NPAD * 4,), jnp.float32),
            pltpu.VMEM((C, 64), jnp.float32),
            pltpu.VMEM((C, 64), jnp.float32),
            pltpu.VMEM((C, ROW), jnp.float32),
            pltpu.VMEM((C, ROW), jnp.float32),
            pltpu.VMEM((C,), jnp.int32),
            pltpu.VMEM((C,), jnp.int32),
            pltpu.VMEM((C,), jnp.int32),
            pltpu.VMEM((C,), jnp.int32),
            pltpu.VMEM((C,), jnp.int32),
            pltpu.VMEM((C,), jnp.int32),
            pltpu.VMEM((C,), jnp.int32),
            pltpu.VMEM((C,), jnp.int32),
            pltpu.VMEM((C,), jnp.int32),
            pltpu.VMEM((C,), jnp.int32),
            pltpu.SemaphoreType.DMA,
            pltpu.SemaphoreType.DMA,
            pltpu.SemaphoreType.DMA,
            pltpu.SemaphoreType.DMA,
            pltpu.SemaphoreType.DMA,
            pltpu.SemaphoreType.DMA,
            pltpu.SemaphoreType.DMA,
            pltpu.SemaphoreType.DMA,
        ],
    )(src, dst, feat0, feat1, elr.reshape(2, NPAD * 4), res_s)

    return jnp.concatenate([out_s[0, :N], out_s[1, :N]], axis=1).reshape(N, H, F)


# speedup vs baseline: 3.1522x; 1.3245x over previous
import jax
import jax.numpy as jnp
from jax import lax
from jax.experimental import pallas as pl
from jax.experimental.pallas import tpu as pltpu
from jax.experimental.pallas import tpu_sc as plsc

N = 10000
NPAD = 10240
E = 640000
D = 128
H = 4
F = 32

NC = 2
NS = 16
C = 80
EPT = E // NS
CPT = EPT // C
RPT = NPAD // NS
FIN = 80
ROW = 80


def _dense_tc(x_ref, g_ref, b_ref, w_ref, alr_ref, rw_ref,
              feat0_ref, feat1_ref, elr_ref, res_ref):
    x = x_ref[:]
    mean = jnp.mean(x, axis=0, keepdims=True)
    var = jnp.mean((x - mean) * (x - mean), axis=0, keepdims=True)
    o = (x - mean) / jnp.sqrt(var + 1e-5) * g_ref[:] + b_ref[:]
    o = jnp.maximum(o, 0.0)
    dn = (((1,), (1,)), ((), ()))
    feat = lax.dot_general(o, w_ref[:], dn, preferred_element_type=jnp.float32)
    feat0_ref[pl.ds(0, N), :] = feat[:, :64]
    feat1_ref[pl.ds(0, N), :] = feat[:, 64:]
    elr8 = lax.dot_general(feat, alr_ref[:], dn,
                           preferred_element_type=jnp.float32)
    elr_ref[0, pl.ds(0, N), :] = elr8[:, :4]
    elr_ref[1, pl.ds(0, N), :] = elr8[:, 4:]
    res = lax.dot_general(o, rw_ref[:], dn, preferred_element_type=jnp.float32)
    res_ref[0, pl.ds(0, N), :] = res[:, :64]
    res_ref[1, pl.ds(0, N), :] = res[:, 64:]


def _edge_sc(src_hbm, dst_hbm, feat0_hbm, feat1_hbm, elr_hbm, res_hbm,
             out_hbm, acc, elr_v, gb0, gb1, sb0, sb1,
             sv0, sv1, sv2, sv3, dv0, dv1, dv2, dv3, sx0, sx1,
             sem_g0, sem_g1, sem_s0, sem_s1,
             sem_i0, sem_i1, sem_i2, sem_i3):
    c = lax.axis_index("c")
    s = lax.axis_index("s")
    gbufs = [gb0, gb1]
    sbufs = [sb0, sb1]
    srcvs = [sv0, sv1, sv2, sv3]
    dstvs = [dv0, dv1, dv2, dv3]
    sdixs = [sx0, sx1]
    sem_gs = [sem_g0, sem_g1]
    sem_ss = [sem_s0, sem_s1]
    sem_is = [sem_i0, sem_i1, sem_i2, sem_i3]

    pltpu.sync_copy(elr_hbm.at[c], elr_v)

    def _zbody(i, carry):
        for g in range(5):
            sb0[i, pl.ds(g * 16, 16)] = jnp.zeros((16,), jnp.float32)
        return carry
    lax.fori_loop(0, C, _zbody, 0)
    for kk in range(8):
        pltpu.sync_copy(sb0.at[pl.ds(0, FIN)],
                        acc.at[pl.ds(s * RPT + kk * FIN, FIN)])
    def _zbody1(i, carry):
        for g in range(5):
            sb1[i, pl.ds(g * 16, 16)] = jnp.zeros((16,), jnp.float32)
        return carry
    lax.fori_loop(0, C, _zbody1, 0)
    plsc.subcore_barrier()

    iot = lax.iota(jnp.int32, 16)
    tbase = s * EPT

    def idx_issue(xv, slot):
        base = tbase + xv * C
        pltpu.async_copy(src_hbm.at[pl.ds(base, C)], srcvs[slot],
                         sem_is[slot])
        pltpu.async_copy(dst_hbm.at[pl.ds(base, C)], dstvs[slot],
                         sem_is[slot])

    def idx_wait(slot):
        pltpu.make_async_copy(src_hbm.at[pl.ds(0, C)], srcvs[slot],
                              sem_is[slot]).wait()
        pltpu.make_async_copy(dst_hbm.at[pl.ds(0, C)], dstvs[slot],
                              sem_is[slot]).wait()

    def gather_issue(slot, p):
        @pl.when(c == 0)
        def _():
            pltpu.async_copy(feat0_hbm.at[srcvs[slot]], gbufs[p], sem_gs[p])

        @pl.when(c == 1)
        def _():
            pltpu.async_copy(feat1_hbm.at[srcvs[slot]], gbufs[p], sem_gs[p])

    def gather_wait(slot, p):
        @pl.when(c == 0)
        def _():
            pltpu.make_async_copy(feat0_hbm.at[srcvs[slot]], gbufs[p],
                                  sem_gs[p]).wait()

        @pl.when(c == 1)
        def _():
            pltpu.make_async_copy(feat1_hbm.at[srcvs[slot]], gbufs[p],
                                  sem_gs[p]).wait()

    def scatter_issue(p):
        pltpu.async_copy(sbufs[p], acc.at[sdixs[p]], sem_ss[p], add=True)

    def scatter_wait(p):
        pltpu.make_async_copy(sbufs[p], acc.at[sdixs[p]], sem_ss[p]).wait()

    def body(xv, slot, p, s_wait):
        gather_wait(slot, p)
        if s_wait:
            scatter_wait(p)

        @plsc.parallel_loop(0, C, step=16, unroll=2)
        def _jbody(off):
            sv = srcvs[slot][pl.ds(off, 16)]
            dv = dstvs[slot][pl.ds(off, 16)]
            wvs = []
            for hl in range(2):
                a = plsc.load_gather(elr_v, [sv * 4 + hl])
                b = plsc.load_gather(elr_v, [dv * 4 + (2 + hl)])
                e = a + b
                e = jnp.where(e >= 0.0, e, 0.2 * e)
                wvs.append(jnp.exp(e))
            sdixs[p][pl.ds(off, 16)] = dv
            for l in range(16):
                row = off + l
                lful = jnp.full((16,), l, jnp.int32)
                w0b = jnp.take_along_axis(wvs[0], lful, axis=0)
                w1b = jnp.take_along_axis(wvs[1], lful, axis=0)
                for g in range(4):
                    wvb = w0b if g < 2 else w1b
                    sbufs[p][row, pl.ds(g * 16, 16)] = (
                        gbufs[p][row, pl.ds(g * 16, 16)] * wvb)
                w01 = jnp.where(iot == 0, w0b,
                                jnp.where(iot == 1, w1b, 0.0))
                sbufs[p][row, pl.ds(64, 16)] = w01
        scatter_issue(p)

    idx_issue(0, 0)
    idx_issue(1, 1)
    idx_issue(2, 2)
    idx_issue(3, 3)
    idx_wait(0)
    gather_issue(0, 0)
    idx_wait(1)
    gather_issue(1, 1)

    for x0 in (0, 1):
        body(x0, x0, x0, False)
        idx_issue(x0 + 4, x0)
        idx_wait(x0 + 2)
        gather_issue(x0 + 2, x0)

    def _quad(t, carry):
        for j in range(4):
            xv = 2 + 4 * t + j
            slot = (2 + j) % 4
            p = j % 2
            body(xv, slot, p, True)

            @pl.when(xv + 4 < CPT)
            def _():
                idx_issue(xv + 4, slot)
            idx_wait((slot + 2) % 4)
            gather_issue((slot + 2) % 4, p)
        return carry
    lax.fori_loop(0, (CPT - 4) // 4, _quad, 0)

    body(CPT - 2, 2, 0, True)
    body(CPT - 1, 3, 1, True)
    scatter_wait(0)
    scatter_wait(1)
    plsc.subcore_barrier()

    for kk in range(8):
        row0 = s * RPT + kk * FIN
        pltpu.sync_copy(acc.at[pl.ds(row0, FIN)], sb0.at[pl.ds(0, FIN)])
        pltpu.sync_copy(res_hbm.at[c, pl.ds(row0, FIN)],
                        gb0.at[pl.ds(0, FIN)])

        def _fin(i, carry):
            wv2 = sb0[i, pl.ds(64, 16)]
            d0 = wv2[0] + 1e-16
            d1 = wv2[1] + 1e-16
            for g in range(4):
                den = jnp.full((16,), d0 if g < 2 else d1, jnp.float32)
                v = sb0[i, pl.ds(g * 16, 16)] / den + gb0[i, pl.ds(g * 16, 16)]
                gb0[i, pl.ds(g * 16, 16)] = v
            return carry
        lax.fori_loop(0, FIN, _fin, 0)
        pltpu.sync_copy(gb0.at[pl.ds(0, FIN)], out_hbm.at[c, pl.ds(row0, FIN)])


def kernel(x, edge_index, bn_gamma, bn_beta, W, attn_l, attn_r, res_W):
    src = edge_index[0]
    dst = edge_index[1]

    alr = jnp.zeros((8, H * F), jnp.float32)
    for cc in range(2):
        for hl in range(2):
            h = 2 * cc + hl
            alr = alr.at[4 * cc + hl, h * F:(h + 1) * F].set(attn_l[h])
            alr = alr.at[4 * cc + 2 + hl, h * F:(h + 1) * F].set(attn_r[h])

    feat0, feat1, elr, res_s = pl.pallas_call(
        _dense_tc,
        out_shape=[
            jax.ShapeDtypeStruct((NPAD, 64), jnp.float32),
            jax.ShapeDtypeStruct((NPAD, 64), jnp.float32),
            jax.ShapeDtypeStruct((2, NPAD, 4), jnp.float32),
            jax.ShapeDtypeStruct((2, NPAD, 64), jnp.float32),
        ],
    )(x, bn_gamma, bn_beta, W, alr, res_W)

    mesh = plsc.VectorSubcoreMesh(core_axis_name="c", subcore_axis_name="s")
    out_s = pl.kernel(
        _edge_sc,
        out_type=jax.ShapeDtypeStruct((2, NPAD, 64), jnp.float32),
        mesh=mesh,
        compiler_params=pltpu.CompilerParams(needs_layout_passes=False,
                                             use_tc_tiling_on_sc=False),
        scratch_types=[
            pltpu.VMEM_SHARED((NPAD, ROW), jnp.float32),
            pltpu.VMEM((NPAD * 4,), jnp.float32),
            pltpu.VMEM((C, 64), jnp.float32),
            pltpu.VMEM((C, 64), jnp.float32),
            pltpu.VMEM((C, ROW), jnp.float32),
            pltpu.VMEM((C, ROW), jnp.float32),
            pltpu.VMEM((C,), jnp.int32),
            pltpu.VMEM((C,), jnp.int32),
            pltpu.VMEM((C,), jnp.int32),
            pltpu.VMEM((C,), jnp.int32),
            pltpu.VMEM((C,), jnp.int32),
            pltpu.VMEM((C,), jnp.int32),
            pltpu.VMEM((C,), jnp.int32),
            pltpu.VMEM((C,), jnp.int32),
            pltpu.VMEM((C,), jnp.int32),
            pltpu.VMEM((C,), jnp.int32),
            pltpu.SemaphoreType.DMA,
            pltpu.SemaphoreType.DMA,
            pltpu.SemaphoreType.DMA,
            pltpu.SemaphoreType.DMA,
            pltpu.SemaphoreType.DMA,
            pltpu.SemaphoreType.DMA,
            pltpu.SemaphoreType.DMA,
            pltpu.SemaphoreType.DMA,
        ],
    )(src, dst, feat0, feat1, elr.reshape(2, NPAD * 4), res_s)

    return jnp.concatenate([out_s[0, :N], out_s[1, :N]], axis=1).reshape(N, H, F)


# speedup vs baseline: 4.1031x; 1.3016x over previous
import jax
import jax.numpy as jnp
from jax import lax
from jax.experimental import pallas as pl
from jax.experimental.pallas import tpu as pltpu
from jax.experimental.pallas import tpu_sc as plsc

N = 10000
NPAD = 10240
E = 640000
D = 128
H = 4
F = 32

NC = 2
NS = 16
C = 80
EPT = E // NS
CPT = EPT // C
RPT = NPAD // NS
FIN = 80
ROW = 80


def _dense_tc(x_ref, g_ref, b_ref, w_ref, alr_ref, rw_ref,
              feat0_ref, feat1_ref, elr_ref, res_ref):
    x = x_ref[:]
    mean = jnp.mean(x, axis=0, keepdims=True)
    var = jnp.mean((x - mean) * (x - mean), axis=0, keepdims=True)
    o = (x - mean) / jnp.sqrt(var + 1e-5) * g_ref[:] + b_ref[:]
    o = jnp.maximum(o, 0.0)
    dn = (((1,), (1,)), ((), ()))
    feat = lax.dot_general(o, w_ref[:], dn, preferred_element_type=jnp.float32)
    feat0_ref[pl.ds(0, N), :] = feat[:, :64]
    feat1_ref[pl.ds(0, N), :] = feat[:, 64:]
    elr8 = lax.dot_general(feat, alr_ref[:], dn,
                           preferred_element_type=jnp.float32)
    elr_ref[0, pl.ds(0, N), :] = elr8[:, :4]
    elr_ref[1, pl.ds(0, N), :] = elr8[:, 4:]
    res = lax.dot_general(o, rw_ref[:], dn, preferred_element_type=jnp.float32)
    res_ref[0, pl.ds(0, N), :] = res[:, :64]
    res_ref[1, pl.ds(0, N), :] = res[:, 64:]


def _edge_sc(src_hbm, dst_hbm, feat0_hbm, feat1_hbm, elr_hbm, res_hbm,
             out_hbm, acc, elr_v, gb0, gb1, sb0, sb1,
             sv0, sv1, sv2, sv3, dv0, dv1, dv2, dv3, sx0, sx1,
             sem_g0, sem_g1, sem_s0, sem_s1,
             sem_i0, sem_i1, sem_i2, sem_i3):
    c = lax.axis_index("c")
    s = lax.axis_index("s")
    gbufs = [gb0, gb1]
    sbufs = [sb0, sb1]
    srcvs = [sv0, sv1, sv2, sv3]
    dstvs = [dv0, dv1, dv2, dv3]
    sdixs = [sx0, sx1]
    sem_gs = [sem_g0, sem_g1]
    sem_ss = [sem_s0, sem_s1]
    sem_is = [sem_i0, sem_i1, sem_i2, sem_i3]

    pltpu.sync_copy(elr_hbm.at[c], elr_v)

    def _zbody(i, carry):
        for g in range(5):
            sb0[i, pl.ds(g * 16, 16)] = jnp.zeros((16,), jnp.float32)
        return carry
    lax.fori_loop(0, C, _zbody, 0)
    for kk in range(8):
        pltpu.sync_copy(sb0.at[pl.ds(0, FIN)],
                        acc.at[pl.ds(s * RPT + kk * FIN, FIN)])
    def _zbody1(i, carry):
        for g in range(5):
            sb1[i, pl.ds(g * 16, 16)] = jnp.zeros((16,), jnp.float32)
        return carry
    lax.fori_loop(0, C, _zbody1, 0)
    plsc.subcore_barrier()

    iot = lax.iota(jnp.int32, 16)
    tbase = s * EPT

    def idx_issue(xv, slot):
        base = tbase + xv * C
        pltpu.async_copy(src_hbm.at[pl.ds(base, C)], srcvs[slot],
                         sem_is[slot])
        pltpu.async_copy(dst_hbm.at[pl.ds(base, C)], dstvs[slot],
                         sem_is[slot])

    def idx_wait(slot):
        pltpu.make_async_copy(src_hbm.at[pl.ds(0, C)], srcvs[slot],
                              sem_is[slot]).wait()
        pltpu.make_async_copy(dst_hbm.at[pl.ds(0, C)], dstvs[slot],
                              sem_is[slot]).wait()

    def gather_issue(slot, p):
        @pl.when(c == 0)
        def _():
            pltpu.async_copy(feat0_hbm.at[srcvs[slot]], gbufs[p], sem_gs[p])

        @pl.when(c == 1)
        def _():
            pltpu.async_copy(feat1_hbm.at[srcvs[slot]], gbufs[p], sem_gs[p])

    def gather_wait(slot, p):
        @pl.when(c == 0)
        def _():
            pltpu.make_async_copy(feat0_hbm.at[srcvs[slot]], gbufs[p],
                                  sem_gs[p]).wait()

        @pl.when(c == 1)
        def _():
            pltpu.make_async_copy(feat1_hbm.at[srcvs[slot]], gbufs[p],
                                  sem_gs[p]).wait()

    def scatter_issue(p):
        pltpu.async_copy(sbufs[p], acc.at[sdixs[p]], sem_ss[p], add=True)

    def scatter_wait(p):
        pltpu.make_async_copy(sbufs[p], acc.at[sdixs[p]], sem_ss[p]).wait()

    def body(xv, slot, p, s_wait):
        gather_wait(slot, p)
        if s_wait:
            scatter_wait(p)

        @plsc.parallel_loop(0, C, step=16, unroll=4)
        def _jbody(off):
            sv = srcvs[slot][pl.ds(off, 16)]
            dv = dstvs[slot][pl.ds(off, 16)]
            wvs = []
            for hl in range(2):
                a = plsc.load_gather(elr_v, [sv * 4 + hl])
                b = plsc.load_gather(elr_v, [dv * 4 + (2 + hl)])
                e = a + b
                e = jnp.where(e >= 0.0, e, 0.2 * e)
                wvs.append(jnp.exp(e))
            sdixs[p][pl.ds(off, 16)] = dv
            for l in range(16):
                row = off + l
                lful = jnp.full((16,), l, jnp.int32)
                w0b = jnp.take_along_axis(wvs[0], lful, axis=0)
                w1b = jnp.take_along_axis(wvs[1], lful, axis=0)
                for g in range(4):
                    wvb = w0b if g < 2 else w1b
                    sbufs[p][row, pl.ds(g * 16, 16)] = (
                        gbufs[p][row, pl.ds(g * 16, 16)] * wvb)
                w01 = jnp.where(iot == 0, w0b,
                                jnp.where(iot == 1, w1b, 0.0))
                sbufs[p][row, pl.ds(64, 16)] = w01
        scatter_issue(p)

    idx_issue(0, 0)
    idx_issue(1, 1)
    idx_issue(2, 2)
    idx_issue(3, 3)
    idx_wait(0)
    gather_issue(0, 0)
    idx_wait(1)
    gather_issue(1, 1)

    for x0 in (0, 1):
        body(x0, x0, x0, False)
        idx_issue(x0 + 4, x0)
        idx_wait(x0 + 2)
        gather_issue(x0 + 2, x0)

    def _quad(t, carry):
        for j in range(4):
            xv = 2 + 4 * t + j
            slot = (2 + j) % 4
            p = j % 2
            body(xv, slot, p, True)

            @pl.when(xv + 4 < CPT)
            def _():
                idx_issue(xv + 4, slot)
            idx_wait((slot + 2) % 4)
            gather_issue((slot + 2) % 4, p)
        return carry
    lax.fori_loop(0, (CPT - 4) // 4, _quad, 0)

    body(CPT - 2, 2, 0, True)
    body(CPT - 1, 3, 1, True)
    scatter_wait(0)
    scatter_wait(1)
    plsc.subcore_barrier()

    for kk in range(8):
        row0 = s * RPT + kk * FIN
        pltpu.sync_copy(acc.at[pl.ds(row0, FIN)], sb0.at[pl.ds(0, FIN)])
        pltpu.sync_copy(res_hbm.at[c, pl.ds(row0, FIN)],
                        gb0.at[pl.ds(0, FIN)])

        def _fin(i, carry):
            wv2 = sb0[i, pl.ds(64, 16)]
            d0 = wv2[0] + 1e-16
            d1 = wv2[1] + 1e-16
            for g in range(4):
                den = jnp.full((16,), d0 if g < 2 else d1, jnp.float32)
                v = sb0[i, pl.ds(g * 16, 16)] / den + gb0[i, pl.ds(g * 16, 16)]
                gb0[i, pl.ds(g * 16, 16)] = v
            return carry
        lax.fori_loop(0, FIN, _fin, 0)
        pltpu.sync_copy(gb0.at[pl.ds(0, FIN)], out_hbm.at[c, pl.ds(row0, FIN)])


def kernel(x, edge_index, bn_gamma, bn_beta, W, attn_l, attn_r, res_W):
    src = edge_index[0]
    dst = edge_index[1]

    alr = jnp.zeros((8, H * F), jnp.float32)
    for cc in range(2):
        for hl in range(2):
            h = 2 * cc + hl
            alr = alr.at[4 * cc + hl, h * F:(h + 1) * F].set(attn_l[h])
            alr = alr.at[4 * cc + 2 + hl, h * F:(h + 1) * F].set(attn_r[h])

    feat0, feat1, elr, res_s = pl.pallas_call(
        _dense_tc,
        out_shape=[
            jax.ShapeDtypeStruct((NPAD, 64), jnp.float32),
            jax.ShapeDtypeStruct((NPAD, 64), jnp.float32),
            jax.ShapeDtypeStruct((2, NPAD, 4), jnp.float32),
            jax.ShapeDtypeStruct((2, NPAD, 64), jnp.float32),
        ],
    )(x, bn_gamma, bn_beta, W, alr, res_W)

    mesh = plsc.VectorSubcoreMesh(core_axis_name="c", subcore_axis_name="s")
    out_s = pl.kernel(
        _edge_sc,
        out_type=jax.ShapeDtypeStruct((2, NPAD, 64), jnp.float32),
        mesh=mesh,
        compiler_params=pltpu.CompilerParams(needs_layout_passes=False,
                                             use_tc_tiling_on_sc=False),
        scratch_types=[
            pltpu.VMEM_SHARED((NPAD, ROW), jnp.float32),
            pltpu.VMEM((NPAD * 4,), jnp.float32),
            pltpu.VMEM((C, 64), jnp.float32),
            pltpu.VMEM((C, 64), jnp.float32),
            pltpu.VMEM((C, ROW), jnp.float32),
            pltpu.VMEM((C, ROW), jnp.float32),
            pltpu.VMEM((C,), jnp.int32),
            pltpu.VMEM((C,), jnp.int32),
            pltpu.VMEM((C,), jnp.int32),
            pltpu.VMEM((C,), jnp.int32),
            pltpu.VMEM((C,), jnp.int32),
            pltpu.VMEM((C,), jnp.int32),
            pltpu.VMEM((C,), jnp.int32),
            pltpu.VMEM((C,), jnp.int32),
            pltpu.VMEM((C,), jnp.int32),
            pltpu.VMEM((C,), jnp.int32),
            pltpu.SemaphoreType.DMA,
            pltpu.SemaphoreType.DMA,
            pltpu.SemaphoreType.DMA,
            pltpu.SemaphoreType.DMA,
            pltpu.SemaphoreType.DMA,
            pltpu.SemaphoreType.DMA,
            pltpu.SemaphoreType.DMA,
            pltpu.SemaphoreType.DMA,
        ],
    )(src, dst, feat0, feat1, elr.reshape(2, NPAD * 4), res_s)

    return jnp.concatenate([out_s[0, :N], out_s[1, :N]], axis=1).reshape(N, H, F)


# speedup vs baseline: 4.2951x; 1.0468x over previous
import jax
import jax.numpy as jnp
from jax import lax
from jax.experimental import pallas as pl
from jax.experimental.pallas import tpu as pltpu
from jax.experimental.pallas import tpu_sc as plsc

N = 10000
NPAD = 10240
E = 640000
D = 128
H = 4
F = 32

NC = 2
NS = 16
C = 80
EPT = E // NS
CPT = EPT // C
RPT = NPAD // NS
FIN = 80
ROW = 80


def _dense_tc(x_ref, g_ref, b_ref, w_ref, alr_ref, rw_ref,
              feat0_ref, feat1_ref, elr_ref, res_ref):
    x = x_ref[:]
    mean = jnp.mean(x, axis=0, keepdims=True)
    var = jnp.mean((x - mean) * (x - mean), axis=0, keepdims=True)
    o = (x - mean) / jnp.sqrt(var + 1e-5) * g_ref[:] + b_ref[:]
    o = jnp.maximum(o, 0.0)
    dn = (((1,), (1,)), ((), ()))
    feat = lax.dot_general(o, w_ref[:], dn, preferred_element_type=jnp.float32)
    feat0_ref[pl.ds(0, N), :] = feat[:, :64]
    feat1_ref[pl.ds(0, N), :] = feat[:, 64:]
    elr8 = lax.dot_general(feat, alr_ref[:], dn,
                           preferred_element_type=jnp.float32)
    elr_ref[0, pl.ds(0, N), :] = elr8[:, :4]
    elr_ref[1, pl.ds(0, N), :] = elr8[:, 4:]
    res = lax.dot_general(o, rw_ref[:], dn, preferred_element_type=jnp.float32)
    res_ref[0, pl.ds(0, N), :] = res[:, :64]
    res_ref[1, pl.ds(0, N), :] = res[:, 64:]


def _edge_sc(src_hbm, dst_hbm, feat0_hbm, feat1_hbm, elr_hbm, res_hbm,
             out_hbm, acc, elr_v, gb0, gb1, sb0, sb1,
             sv0, sv1, sv2, sv3, dv0, dv1, dv2, dv3, sx0, sx1,
             sem_g0, sem_g1, sem_s0, sem_s1,
             sem_i0, sem_i1, sem_i2, sem_i3):
    c = lax.axis_index("c")
    s = lax.axis_index("s")
    gbufs = [gb0, gb1]
    sbufs = [sb0, sb1]
    srcvs = [sv0, sv1, sv2, sv3]
    dstvs = [dv0, dv1, dv2, dv3]
    sdixs = [sx0, sx1]
    sem_gs = [sem_g0, sem_g1]
    sem_ss = [sem_s0, sem_s1]
    sem_is = [sem_i0, sem_i1, sem_i2, sem_i3]

    pltpu.sync_copy(elr_hbm.at[c], elr_v)

    def _zbody(i, carry):
        for g in range(5):
            sb0[i, pl.ds(g * 16, 16)] = jnp.zeros((16,), jnp.float32)
        return carry
    lax.fori_loop(0, C, _zbody, 0)
    for kk in range(8):
        pltpu.sync_copy(sb0.at[pl.ds(0, FIN)],
                        acc.at[pl.ds(s * RPT + kk * FIN, FIN)])
    def _zbody1(i, carry):
        for g in range(5):
            sb1[i, pl.ds(g * 16, 16)] = jnp.zeros((16,), jnp.float32)
        return carry
    lax.fori_loop(0, C, _zbody1, 0)
    plsc.subcore_barrier()

    iot = lax.iota(jnp.int32, 16)
    tbase = s * EPT

    def idx_issue(xv, slot):
        base = tbase + xv * C
        pltpu.async_copy(src_hbm.at[pl.ds(base, C)], srcvs[slot],
                         sem_is[slot])
        pltpu.async_copy(dst_hbm.at[pl.ds(base, C)], dstvs[slot],
                         sem_is[slot])

    def idx_wait(slot):
        pltpu.make_async_copy(src_hbm.at[pl.ds(0, C)], srcvs[slot],
                              sem_is[slot]).wait()
        pltpu.make_async_copy(dst_hbm.at[pl.ds(0, C)], dstvs[slot],
                              sem_is[slot]).wait()

    def gather_issue(slot, p):
        @pl.when(c == 0)
        def _():
            pltpu.async_copy(feat0_hbm.at[srcvs[slot]], gbufs[p], sem_gs[p])

        @pl.when(c == 1)
        def _():
            pltpu.async_copy(feat1_hbm.at[srcvs[slot]], gbufs[p], sem_gs[p])

    def gather_wait(slot, p):
        @pl.when(c == 0)
        def _():
            pltpu.make_async_copy(feat0_hbm.at[srcvs[slot]], gbufs[p],
                                  sem_gs[p]).wait()

        @pl.when(c == 1)
        def _():
            pltpu.make_async_copy(feat1_hbm.at[srcvs[slot]], gbufs[p],
                                  sem_gs[p]).wait()

    def scatter_issue(p):
        pltpu.async_copy(sbufs[p], acc.at[sdixs[p]], sem_ss[p], add=True)

    def scatter_wait(p):
        pltpu.make_async_copy(sbufs[p], acc.at[sdixs[p]], sem_ss[p]).wait()

    def body(xv, slot, p, s_wait):
        gather_wait(slot, p)
        if s_wait:
            scatter_wait(p)

        @plsc.parallel_loop(0, C, step=16, unroll=5)
        def _jbody(off):
            sv = srcvs[slot][pl.ds(off, 16)]
            dv = dstvs[slot][pl.ds(off, 16)]
            wvs = []
            for hl in range(2):
                a = plsc.load_gather(elr_v, [sv * 4 + hl])
                b = plsc.load_gather(elr_v, [dv * 4 + (2 + hl)])
                e = a + b
                e = jnp.where(e >= 0.0, e, 0.2 * e)
                wvs.append(jnp.exp(e))
            sdixs[p][pl.ds(off, 16)] = dv
            for l in range(16):
                row = off + l
                lful = jnp.full((16,), l, jnp.int32)
                w0b = jnp.take_along_axis(wvs[0], lful, axis=0)
                w1b = jnp.take_along_axis(wvs[1], lful, axis=0)
                for g in range(4):
                    wvb = w0b if g < 2 else w1b
                    sbufs[p][row, pl.ds(g * 16, 16)] = (
                        gbufs[p][row, pl.ds(g * 16, 16)] * wvb)
                w01 = jnp.where(iot == 0, w0b,
                                jnp.where(iot == 1, w1b, 0.0))
                sbufs[p][row, pl.ds(64, 16)] = w01
        scatter_issue(p)

    idx_issue(0, 0)
    idx_issue(1, 1)
    idx_issue(2, 2)
    idx_issue(3, 3)
    idx_wait(0)
    gather_issue(0, 0)
    idx_wait(1)
    gather_issue(1, 1)

    for x0 in (0, 1):
        body(x0, x0, x0, False)
        idx_issue(x0 + 4, x0)
        idx_wait(x0 + 2)
        gather_issue(x0 + 2, x0)

    def _quad(t, carry):
        for j in range(4):
            xv = 2 + 4 * t + j
            slot = (2 + j) % 4
            p = j % 2
            body(xv, slot, p, True)

            @pl.when(xv + 4 < CPT)
            def _():
                idx_issue(xv + 4, slot)
            idx_wait((slot + 2) % 4)
            gather_issue((slot + 2) % 4, p)
        return carry
    lax.fori_loop(0, (CPT - 4) // 4, _quad, 0)

    body(CPT - 2, 2, 0, True)
    body(CPT - 1, 3, 1, True)
    scatter_wait(0)
    scatter_wait(1)
    plsc.subcore_barrier()

    for kk in range(8):
        row0 = s * RPT + kk * FIN
        pltpu.sync_copy(acc.at[pl.ds(row0, FIN)], sb0.at[pl.ds(0, FIN)])
        pltpu.sync_copy(res_hbm.at[c, pl.ds(row0, FIN)],
                        gb0.at[pl.ds(0, FIN)])

        def _fin(i, carry):
            wv2 = sb0[i, pl.ds(64, 16)]
            d0 = wv2[0] + 1e-16
            d1 = wv2[1] + 1e-16
            for g in range(4):
                den = jnp.full((16,), d0 if g < 2 else d1, jnp.float32)
                v = sb0[i, pl.ds(g * 16, 16)] / den + gb0[i, pl.ds(g * 16, 16)]
                gb0[i, pl.ds(g * 16, 16)] = v
            return carry
        lax.fori_loop(0, FIN, _fin, 0)
        pltpu.sync_copy(gb0.at[pl.ds(0, FIN)], out_hbm.at[c, pl.ds(row0, FIN)])


def kernel(x, edge_index, bn_gamma, bn_beta, W, attn_l, attn_r, res_W):
    src = edge_index[0]
    dst = edge_index[1]

    alr = jnp.zeros((8, H * F), jnp.float32)
    for cc in range(2):
        for hl in range(2):
            h = 2 * cc + hl
            alr = alr.at[4 * cc + hl, h * F:(h + 1) * F].set(attn_l[h])
            alr = alr.at[4 * cc + 2 + hl, h * F:(h + 1) * F].set(attn_r[h])

    feat0, feat1, elr, res_s = pl.pallas_call(
        _dense_tc,
        out_shape=[
            jax.ShapeDtypeStruct((NPAD, 64), jnp.float32),
            jax.ShapeDtypeStruct((NPAD, 64), jnp.float32),
            jax.ShapeDtypeStruct((2, NPAD, 4), jnp.float32),
            jax.ShapeDtypeStruct((2, NPAD, 64), jnp.float32),
        ],
    )(x, bn_gamma, bn_beta, W, alr, res_W)

    mesh = plsc.VectorSubcoreMesh(core_axis_name="c", subcore_axis_name="s")
    out_s = pl.kernel(
        _edge_sc,
        out_type=jax.ShapeDtypeStruct((2, NPAD, 64), jnp.float32),
        mesh=mesh,
        compiler_params=pltpu.CompilerParams(needs_layout_passes=False,
                                             use_tc_tiling_on_sc=False),
        scratch_types=[
            pltpu.VMEM_SHARED((NPAD, ROW), jnp.float32),
            pltpu.VMEM((NPAD * 4,), jnp.float32),
            pltpu.VMEM((C, 64), jnp.float32),
            pltpu.VMEM((C, 64), jnp.float32),
            pltpu.VMEM((C, ROW), jnp.float32),
            pltpu.VMEM((C, ROW), jnp.float32),
            pltpu.VMEM((C,), jnp.int32),
            pltpu.VMEM((C,), jnp.int32),
            pltpu.VMEM((C,), jnp.int32),
            pltpu.VMEM((C,), jnp.int32),
            pltpu.VMEM((C,), jnp.int32),
            pltpu.VMEM((C,), jnp.int32),
            pltpu.VMEM((C,), jnp.int32),
            pltpu.VMEM((C,), jnp.int32),
            pltpu.VMEM((C,), jnp.int32),
            pltpu.VMEM((C,), jnp.int32),
            pltpu.SemaphoreType.DMA,
            pltpu.SemaphoreType.DMA,
            pltpu.SemaphoreType.DMA,
            pltpu.SemaphoreType.DMA,
            pltpu.SemaphoreType.DMA,
            pltpu.SemaphoreType.DMA,
            pltpu.SemaphoreType.DMA,
            pltpu.SemaphoreType.DMA,
        ],
    )(src, dst, feat0, feat1, elr.reshape(2, NPAD * 4), res_s)

    return jnp.concatenate([out_s[0, :N], out_s[1, :N]], axis=1).reshape(N, H, F)
